# baseline (speedup 1.0000x reference)
import jax
import jax.numpy as jnp
from jax import lax
from jax.experimental import pallas as pl
from jax.experimental.pallas import tpu as pltpu
from jax.experimental.pallas import tpu_sc as plsc

N = 10000
E = 320000
D = 128
DE = 16
NC, NS, L = 2, 16, 16

E_PAD = 327680
R = 2560
PK_PER_TILE = R // (NC * NS)
CH = 32
CHUNKS_PER_TILE = (128 // CH) * PK_PER_TILE
EDGES_PER_TILE = E_PAD // (NC * NS)
ZROWS = 632
ZLAST = N - ZROWS
IDX_SHIFT = 14
NBUF = 4


def _leaky(x):
    return jnp.where(x >= 0, x, x * jnp.float32(0.01))


_BN = 2000


def _mm_node_body(x_ref, w_ref, o_ref):
    o_ref[...] = jnp.dot(x_ref[...], w_ref[...], preferred_element_type=jnp.float32)


_mm_node = pl.pallas_call(
    _mm_node_body,
    grid=(N // _BN,),
    in_specs=[
        pl.BlockSpec((_BN, D), lambda i: (i, 0)),
        pl.BlockSpec((D, D), lambda i: (0, 0)),
    ],
    out_specs=pl.BlockSpec((_BN, D), lambda i: (i, 0)),
    out_shape=jax.ShapeDtypeStruct((N, D), jnp.float32),
)


_BE = 2048


def _edge_body(e_ref, w1_ref, b1_ref, w2_ref, b2_ref, o_ref):
    h = jnp.dot(e_ref[...], w1_ref[...], preferred_element_type=jnp.float32)
    h = _leaky(h + b1_ref[...])
    h = jnp.dot(h, w2_ref[...], preferred_element_type=jnp.float32)
    h = _leaky(h + b2_ref[...])
    row = pl.program_id(0) * _BE + lax.broadcasted_iota(jnp.int32, (_BE, 1), 0)
    o_ref[...] = jnp.where(row < E, h, jnp.float32(0.0))


_edge_proj = pl.pallas_call(
    _edge_body,
    grid=(E_PAD // _BE,),
    in_specs=[
        pl.BlockSpec((_BE, DE), lambda i: (i, 0)),
        pl.BlockSpec((DE, D), lambda i: (0, 0)),
        pl.BlockSpec((1, D), lambda i: (0, 0)),
        pl.BlockSpec((D, D), lambda i: (0, 0)),
        pl.BlockSpec((1, D), lambda i: (0, 0)),
    ],
    out_specs=pl.BlockSpec((_BE, D), lambda i: (i, 0)),
    out_shape=jax.ShapeDtypeStruct((E_PAD, D), jnp.float32),
)


def _sc_body(msg_hbm, h_hbm, pk_hbm, out_hbm, *scr):
    pkv = scr[0]
    idxrv = scr[1]
    segrv = scr[2]
    rows = list(scr[3:3 + NBUF])
    hvb = list(scr[3 + NBUF:3 + 2 * NBUF])
    acc_sh = scr[3 + 2 * NBUF]
    gsem = list(scr[4 + 2 * NBUF:4 + 3 * NBUF])
    hsem = list(scr[4 + 3 * NBUF:4 + 4 * NBUF])
    ssem = list(scr[4 + 4 * NBUF:4 + 5 * NBUF])

    c = lax.axis_index("c")
    s = lax.axis_index("s")
    w = c * NS + s

    def _z(i, _):
        zero = jnp.zeros((L,), jnp.float32)
        for k in range(D // L):
            rows[0][i, pl.ds(k * L, L)] = zero
        return 0
    lax.fori_loop(0, CH, _z, 0)
    zbase = jnp.minimum(s * ZROWS, ZLAST)
    for t in range(ZROWS // CH):
        pltpu.sync_copy(rows[0], acc_sh.at[pl.ds(zbase + t * CH, CH)])
    zrem = ZROWS % CH
    if zrem:
        pltpu.sync_copy(rows[0].at[pl.ds(0, zrem)],
                        acc_sh.at[pl.ds(zbase + (ZROWS // CH) * CH, zrem)])
    plsc.subcore_barrier()

    pltpu.sync_copy(pk_hbm.at[pl.ds(w * PK_PER_TILE, PK_PER_TILE)], pkv)

    ebase = w * EDGES_PER_TILE
    SUB = 128 // CH
    SUB_SHIFT = SUB.bit_length() - 1

    def _prep(t, b):
        off = lax.bitwise_and(t, SUB - 1) * CH
        pkrow = lax.shift_right_logical(t, SUB_SHIFT)

        def _unpack(k, _):
            pk = pkv[pkrow, pl.ds(off + k * L, L)]
            sl = pl.ds(k * L, L)
            idxrv[b, sl] = lax.shift_right_logical(pk, IDX_SHIFT)
            segrv[b, sl] = lax.bitwise_and(pk, (1 << IDX_SHIFT) - 1)
            return 0
        lax.fori_loop(0, CH // L, _unpack, 0)
        pltpu.async_copy(msg_hbm.at[idxrv.at[b]], rows[b], gsem[b])
        pltpu.async_copy(h_hbm.at[pl.ds(ebase + t * CH, CH)],
                         hvb[b], hsem[b])

    for b in range(NBUF):
        _prep(jnp.int32(b), b)

    def _step(g, _):
        for b in range(NBUF):
            t = g * NBUF + b
            pltpu.make_async_copy(msg_hbm.at[idxrv.at[b]],
                                  rows[b], gsem[b]).wait()
            pltpu.make_async_copy(h_hbm.at[pl.ds(ebase, CH)],
                                  hvb[b], hsem[b]).wait()

            def _mul(i, _):
                for k in range(D // L):
                    sl = pl.ds(k * L, L)
                    rows[b][i, sl] = rows[b][i, sl] * hvb[b][i, sl]
                return 0
            lax.fori_loop(0, CH, _mul, 0)
            pltpu.async_copy(rows[b], acc_sh.at[segrv.at[b]],
                             ssem[b], add=True)
            t2 = t + NBUF

            @pl.when(t2 < CHUNKS_PER_TILE)
            def _():
                pltpu.make_async_copy(rows[b], acc_sh.at[segrv.at[b]],
                                      ssem[b]).wait()
                _prep(t2, b)
        return 0
    lax.fori_loop(0, CHUNKS_PER_TILE // NBUF, _step, 0)

    for b in range(NBUF):
        pltpu.make_async_copy(rows[b], acc_sh.at[segrv.at[b]],
                              ssem[b]).wait()
    plsc.subcore_barrier()
    pltpu.sync_copy(acc_sh.at[pl.ds(zbase, ZROWS)],
                    out_hbm.at[c, pl.ds(zbase, ZROWS)])


_sc_gather_scatter = pl.kernel(
    _sc_body,
    out_type=jax.ShapeDtypeStruct((NC, N, D), jnp.float32),
    mesh=plsc.VectorSubcoreMesh(
        core_axis_name="c", subcore_axis_name="s",
        num_cores=NC, num_subcores=NS),
    scratch_types=(
        [pltpu.VMEM((PK_PER_TILE, 128), jnp.int32),
         pltpu.VMEM((NBUF, CH), jnp.int32),
         pltpu.VMEM((NBUF, CH), jnp.int32)]
        + [pltpu.VMEM((CH, D), jnp.float32) for _ in range(2 * NBUF)]
        + [pltpu.VMEM_SHARED((N, D), jnp.float32)]
        + [pltpu.SemaphoreType.DMA for _ in range(3 * NBUF)]
    ),
)


def _psum_body(p_ref, o_ref):
    o_ref[...] = p_ref[0] + p_ref[1]


_psum = pl.pallas_call(
    _psum_body,
    grid=(N // _BN,),
    in_specs=[pl.BlockSpec((NC, _BN, D), lambda i: (0, i, 0))],
    out_specs=pl.BlockSpec((_BN, D), lambda i: (i, 0)),
    out_shape=jax.ShapeDtypeStruct((N, D), jnp.float32),
)


@jax.jit
def kernel(node, edge, seg_i, idx_j, W_node, W_e1, b_e1, W_e2, b_e2):
    msg = _mm_node(node, W_node)
    edge_pad = jnp.pad(edge, ((0, E_PAD - E), (0, 0)))
    h = _edge_proj(edge_pad, W_e1, b_e1.reshape(1, D), W_e2, b_e2.reshape(1, D))
    packed = jnp.pad((idx_j << IDX_SHIFT) | seg_i, (0, E_PAD - E))
    pk2 = packed.reshape(R, 128)
    partials = _sc_gather_scatter(msg, h, pk2)
    return _psum(partials)


# speedup vs baseline: 1.0017x; 1.0017x over previous
import jax
import jax.numpy as jnp
from jax import lax
from jax.experimental import pallas as pl
from jax.experimental.pallas import tpu as pltpu
from jax.experimental.pallas import tpu_sc as plsc

N = 10000
E = 320000
D = 128
DE = 16
NC, NS, L = 2, 16, 16

E_PAD = 327680
R = 2560
PK_ROWS = 4608
PK_PER_TILE = R // (NC * NS)
CH = 64
CHUNKS_PER_TILE = (128 // CH) * PK_PER_TILE
EDGES_PER_TILE = E_PAD // (NC * NS)
ZROWS = 632
ZLAST = N - ZROWS
IDX_SHIFT = 14
NBUF = 3


def _leaky(x):
    return jnp.where(x >= 0, x, x * jnp.float32(0.01))


_BN = 2000


def _mm_node_body(x_ref, w_ref, o_ref):
    o_ref[...] = jnp.dot(x_ref[...], w_ref[...], preferred_element_type=jnp.float32)


_mm_node = pl.pallas_call(
    _mm_node_body,
    grid=(N // _BN,),
    in_specs=[
        pl.BlockSpec((_BN, D), lambda i: (i, 0)),
        pl.BlockSpec((D, D), lambda i: (0, 0)),
    ],
    out_specs=pl.BlockSpec((_BN, D), lambda i: (i, 0)),
    out_shape=jax.ShapeDtypeStruct((N, D), jnp.float32),
)


_BE = 2048


def _edge_body(e_ref, w1_ref, b1_ref, w2_ref, b2_ref, o_ref):
    h = jnp.dot(e_ref[...], w1_ref[...], preferred_element_type=jnp.float32)
    h = _leaky(h + b1_ref[...])
    h = jnp.dot(h, w2_ref[...], preferred_element_type=jnp.float32)
    h = _leaky(h + b2_ref[...])
    row = pl.program_id(0) * _BE + lax.broadcasted_iota(jnp.int32, (_BE, 1), 0)
    o_ref[...] = jnp.where(row < E, h, jnp.float32(0.0))


_edge_proj = pl.pallas_call(
    _edge_body,
    grid=(E_PAD // _BE,),
    in_specs=[
        pl.BlockSpec((_BE, DE), lambda i: (i, 0)),
        pl.BlockSpec((DE, D), lambda i: (0, 0)),
        pl.BlockSpec((1, D), lambda i: (0, 0)),
        pl.BlockSpec((D, D), lambda i: (0, 0)),
        pl.BlockSpec((1, D), lambda i: (0, 0)),
    ],
    out_specs=pl.BlockSpec((_BE, D), lambda i: (i, 0)),
    out_shape=jax.ShapeDtypeStruct((E_PAD, D), jnp.float32),
)


def _sc_body(msg_hbm, h_hbm, pk_hbm, out_hbm, *scr):
    pkr = scr[0]
    idxrv = scr[1]
    segrv = scr[2]
    rows = list(scr[3:3 + NBUF])
    hvb = list(scr[3 + NBUF:3 + 2 * NBUF])
    acc_sh = scr[3 + 2 * NBUF]
    gsem = list(scr[4 + 2 * NBUF:4 + 3 * NBUF])
    hsem = list(scr[4 + 3 * NBUF:4 + 4 * NBUF])
    ssem = list(scr[4 + 4 * NBUF:4 + 5 * NBUF])
    pksem = list(scr[4 + 5 * NBUF:4 + 6 * NBUF])

    c = lax.axis_index("c")
    s = lax.axis_index("s")
    w = c * NS + s

    def _z(i, _):
        zero = jnp.zeros((L,), jnp.float32)
        for k in range(D // L):
            rows[0][i, pl.ds(k * L, L)] = zero
        return 0
    lax.fori_loop(0, CH, _z, 0)
    zbase = jnp.minimum(s * ZROWS, ZLAST)
    for t in range(ZROWS // CH):
        pltpu.sync_copy(rows[0], acc_sh.at[pl.ds(zbase + t * CH, CH)])
    zrem = ZROWS % CH
    if zrem:
        pltpu.sync_copy(rows[0].at[pl.ds(0, zrem)],
                        acc_sh.at[pl.ds(zbase + (ZROWS // CH) * CH, zrem)])
    plsc.subcore_barrier()

    ebase = w * EDGES_PER_TILE

    def _pkload(t, b):
        pltpu.async_copy(pk_hbm.at[pl.ds(ebase + t * CH, CH)],
                         pkr.at[b], pksem[b])

    def _prep(t, b):
        pltpu.make_async_copy(pk_hbm.at[pl.ds(ebase, CH)],
                              pkr.at[b], pksem[b]).wait()

        def _unpack(k, _):
            pk = pkr[b, pl.ds(k * L, L)]
            sl = pl.ds(k * L, L)
            idxrv[b, sl] = lax.shift_right_logical(pk, IDX_SHIFT)
            segrv[b, sl] = lax.bitwise_and(pk, (1 << IDX_SHIFT) - 1)
            return 0
        lax.fori_loop(0, CH // L, _unpack, 0)
        pltpu.async_copy(msg_hbm.at[idxrv.at[b]], rows[b], gsem[b])
        pltpu.async_copy(h_hbm.at[pl.ds(ebase + t * CH, CH)],
                         hvb[b], hsem[b])

        @pl.when(t + NBUF < CHUNKS_PER_TILE)
        def _():
            _pkload(t + NBUF, b)

    for b in range(NBUF):
        _pkload(jnp.int32(b), b)
    for b in range(NBUF):
        _prep(jnp.int32(b), b)

    def _work(t, b):
        pltpu.make_async_copy(msg_hbm.at[idxrv.at[b]],
                              rows[b], gsem[b]).wait()
        pltpu.make_async_copy(h_hbm.at[pl.ds(ebase, CH)],
                              hvb[b], hsem[b]).wait()

        def _mul(i, _):
            for k in range(D // L):
                sl = pl.ds(k * L, L)
                rows[b][i, sl] = rows[b][i, sl] * hvb[b][i, sl]
            return 0
        lax.fori_loop(0, CH, _mul, 0)
        pltpu.async_copy(rows[b], acc_sh.at[segrv.at[b]],
                         ssem[b], add=True)
        t2 = t + NBUF

        @pl.when(t2 < CHUNKS_PER_TILE)
        def _():
            pltpu.make_async_copy(rows[b], acc_sh.at[segrv.at[b]],
                                  ssem[b]).wait()
            _prep(t2, b)

    def _step(g, _):
        for b in range(NBUF):
            _work(g * NBUF + b, b)
        return 0
    _MAIN = CHUNKS_PER_TILE // NBUF
    lax.fori_loop(0, _MAIN, _step, 0)
    for i in range(CHUNKS_PER_TILE % NBUF):
        t = _MAIN * NBUF + i
        _work(jnp.int32(t), t % NBUF)

    for b in range(NBUF):
        pltpu.make_async_copy(rows[b], acc_sh.at[segrv.at[b]],
                              ssem[b]).wait()
    plsc.subcore_barrier()
    pltpu.sync_copy(acc_sh.at[pl.ds(zbase, ZROWS)],
                    out_hbm.at[c, pl.ds(zbase, ZROWS)])


_sc_gather_scatter = pl.kernel(
    _sc_body,
    out_type=jax.ShapeDtypeStruct((NC, N, D), jnp.float32),
    mesh=plsc.VectorSubcoreMesh(
        core_axis_name="c", subcore_axis_name="s",
        num_cores=NC, num_subcores=NS),
    scratch_types=(
        [pltpu.VMEM((NBUF, CH), jnp.int32),
         pltpu.VMEM((NBUF, CH), jnp.int32),
         pltpu.VMEM((NBUF, CH), jnp.int32)]
        + [pltpu.VMEM((CH, D), jnp.float32) for _ in range(2 * NBUF)]
        + [pltpu.VMEM_SHARED((N, D), jnp.float32)]
        + [pltpu.SemaphoreType.DMA for _ in range(4 * NBUF)]
    ),
)


def _psum_body(p_ref, o_ref):
    o_ref[...] = p_ref[0] + p_ref[1]


_psum = pl.pallas_call(
    _psum_body,
    grid=(N // _BN,),
    in_specs=[pl.BlockSpec((NC, _BN, D), lambda i: (0, i, 0))],
    out_specs=pl.BlockSpec((_BN, D), lambda i: (i, 0)),
    out_shape=jax.ShapeDtypeStruct((N, D), jnp.float32),
)


@jax.jit
def kernel(node, edge, seg_i, idx_j, W_node, W_e1, b_e1, W_e2, b_e2):
    msg = _mm_node(node, W_node)
    edge_pad = jnp.pad(edge, ((0, E_PAD - E), (0, 0)))
    h = _edge_proj(edge_pad, W_e1, b_e1.reshape(1, D), W_e2, b_e2.reshape(1, D))
    pk1 = jnp.pad((idx_j << IDX_SHIFT) | seg_i, (0, E_PAD - E))
    partials = _sc_gather_scatter(msg, h, pk1)
    return _psum(partials)


# speedup vs baseline: 1.0436x; 1.0419x over previous
import jax
import jax.numpy as jnp
from jax import lax
from jax.experimental import pallas as pl
from jax.experimental.pallas import tpu as pltpu
from jax.experimental.pallas import tpu_sc as plsc

N = 10000
E = 320000
D = 128
DE = 16
NC, NS, L = 2, 16, 16

E_PAD = 327680
R = 2560
PK_PER_TILE = R // (NC * NS)
CH = 64
CHUNKS_PER_TILE = 2 * PK_PER_TILE
EDGES_PER_TILE = E_PAD // (NC * NS)
ZROWS = 632
ZLAST = N - ZROWS
IDX_SHIFT = 14
NBUF = 2


def _leaky(x):
    return jnp.where(x >= 0, x, x * jnp.float32(0.01))


_BN = 2000


def _mm_node_body(x_ref, w_ref, o_ref):
    o_ref[...] = jnp.dot(x_ref[...], w_ref[...], preferred_element_type=jnp.float32)


_mm_node = pl.pallas_call(
    _mm_node_body,
    grid=(N // _BN,),
    in_specs=[
        pl.BlockSpec((_BN, D), lambda i: (i, 0)),
        pl.BlockSpec((D, D), lambda i: (0, 0)),
    ],
    out_specs=pl.BlockSpec((_BN, D), lambda i: (i, 0)),
    out_shape=jax.ShapeDtypeStruct((N, D), jnp.float32),
)


_BE = 2048


def _edge_body(e_ref, w1_ref, b1_ref, w2_ref, b2_ref, o_ref):
    h = jnp.dot(e_ref[...], w1_ref[...], preferred_element_type=jnp.float32)
    h = _leaky(h + b1_ref[...])
    h = jnp.dot(h, w2_ref[...], preferred_element_type=jnp.float32)
    h = _leaky(h + b2_ref[...])
    row = pl.program_id(0) * _BE + lax.broadcasted_iota(jnp.int32, (_BE, 1), 0)
    o_ref[...] = jnp.where(row < E, h, jnp.float32(0.0))


_edge_proj = pl.pallas_call(
    _edge_body,
    grid=(E_PAD // _BE,),
    in_specs=[
        pl.BlockSpec((_BE, DE), lambda i: (i, 0)),
        pl.BlockSpec((DE, D), lambda i: (0, 0)),
        pl.BlockSpec((1, D), lambda i: (0, 0)),
        pl.BlockSpec((D, D), lambda i: (0, 0)),
        pl.BlockSpec((1, D), lambda i: (0, 0)),
    ],
    out_specs=pl.BlockSpec((_BE, D), lambda i: (i, 0)),
    out_shape=jax.ShapeDtypeStruct((E_PAD, D), jnp.float32),
)


def _sc_body(msg_hbm, h_hbm, pk_hbm, out_hbm,
             pkv, idxrv, segrv, rows0, rows1, hv0, hv1, acc_sh,
             g0, g1, gb0, gb1, hs0, hs1, ss0, ss1):
    c = lax.axis_index("c")
    s = lax.axis_index("s")
    w = c * NS + s
    gsem, gsemB, hsem, ssem = [g0, g1], [gb0, gb1], [hs0, hs1], [ss0, ss1]
    rows, hvb = [rows0, rows1], [hv0, hv1]

    def _z(i, _):
        zero = jnp.zeros((L,), jnp.float32)
        for k in range(D // L):
            rows0[i, pl.ds(k * L, L)] = zero
            rows1[i, pl.ds(k * L, L)] = zero
        return 0
    lax.fori_loop(0, CH, _z, 0)
    zbase = jnp.minimum(s * ZROWS, ZLAST)
    nz = ZROWS // (2 * CH)
    for t in range(nz):
        pltpu.sync_copy(rows0, acc_sh.at[pl.ds(zbase + (2 * t) * CH, CH)])
        pltpu.sync_copy(rows1, acc_sh.at[pl.ds(zbase + (2 * t + 1) * CH, CH)])
    zrem = ZROWS - nz * 2 * CH
    if zrem:
        pltpu.sync_copy(rows0.at[pl.ds(0, zrem)],
                        acc_sh.at[pl.ds(zbase + nz * 2 * CH, zrem)])
    plsc.subcore_barrier()

    pltpu.sync_copy(pk_hbm.at[pl.ds(w * PK_PER_TILE, PK_PER_TILE)], pkv)

    ebase = w * EDGES_PER_TILE

    def _prep(t, b):
        half = lax.bitwise_and(t, 1) * CH
        pkrow = lax.shift_right_logical(t, 1)

        def _unpack(k, _):
            pk = pkv[pkrow, pl.ds(half + k * L, L)]
            sl = pl.ds(k * L, L)
            idxrv[b, sl] = lax.shift_right_logical(pk, IDX_SHIFT)
            segrv[b, sl] = lax.bitwise_and(pk, (1 << IDX_SHIFT) - 1)
            return 0
        lax.fori_loop(0, CH // L, _unpack, 0)
        GH = CH // 2
        pltpu.async_copy(msg_hbm.at[idxrv.at[b, pl.ds(0, GH)]],
                         rows[b].at[pl.ds(0, GH)], gsem[b])
        pltpu.async_copy(msg_hbm.at[idxrv.at[b, pl.ds(GH, GH)]],
                         rows[b].at[pl.ds(GH, GH)], gsemB[b])
        pltpu.async_copy(h_hbm.at[pl.ds(ebase + t * CH, CH)],
                         hvb[b], hsem[b])

    for b in range(NBUF):
        _prep(jnp.int32(b), b)

    def _step(g, _):
        for b in range(NBUF):
            t = g * NBUF + b
            GH = CH // 2
            pltpu.make_async_copy(msg_hbm.at[idxrv.at[b, pl.ds(0, GH)]],
                                  rows[b].at[pl.ds(0, GH)], gsem[b]).wait()
            pltpu.make_async_copy(msg_hbm.at[idxrv.at[b, pl.ds(GH, GH)]],
                                  rows[b].at[pl.ds(GH, GH)], gsemB[b]).wait()
            pltpu.make_async_copy(h_hbm.at[pl.ds(ebase, CH)],
                                  hvb[b], hsem[b]).wait()

            def _mul(i, _):
                for k in range(D // L):
                    sl = pl.ds(k * L, L)
                    rows[b][i, sl] = rows[b][i, sl] * hvb[b][i, sl]
                return 0
            lax.fori_loop(0, CH, _mul, 0)
            pltpu.async_copy(rows[b], acc_sh.at[segrv.at[b]],
                             ssem[b], add=True)
            t2 = t + NBUF

            @pl.when(t2 < CHUNKS_PER_TILE)
            def _():
                pltpu.make_async_copy(rows[b], acc_sh.at[segrv.at[b]],
                                      ssem[b]).wait()
                _prep(t2, b)
        return 0
    lax.fori_loop(0, CHUNKS_PER_TILE // NBUF, _step, 0)

    for b in range(NBUF):
        pltpu.make_async_copy(rows[b], acc_sh.at[segrv.at[b]],
                              ssem[b]).wait()
    plsc.subcore_barrier()
    pltpu.sync_copy(acc_sh.at[pl.ds(zbase, ZROWS)],
                    out_hbm.at[c, pl.ds(zbase, ZROWS)])


_sc_gather_scatter = pl.kernel(
    _sc_body,
    out_type=jax.ShapeDtypeStruct((NC, N, D), jnp.float32),
    mesh=plsc.VectorSubcoreMesh(
        core_axis_name="c", subcore_axis_name="s",
        num_cores=NC, num_subcores=NS),
    scratch_types=[
        pltpu.VMEM((PK_PER_TILE, 2 * CH), jnp.int32),
        pltpu.VMEM((NBUF, CH), jnp.int32),
        pltpu.VMEM((NBUF, CH), jnp.int32),
        pltpu.VMEM((CH, D), jnp.float32),
        pltpu.VMEM((CH, D), jnp.float32),
        pltpu.VMEM((CH, D), jnp.float32),
        pltpu.VMEM((CH, D), jnp.float32),
        pltpu.VMEM_SHARED((N, D), jnp.float32),
        pltpu.SemaphoreType.DMA,
        pltpu.SemaphoreType.DMA,
        pltpu.SemaphoreType.DMA,
        pltpu.SemaphoreType.DMA,
        pltpu.SemaphoreType.DMA,
        pltpu.SemaphoreType.DMA,
        pltpu.SemaphoreType.DMA,
        pltpu.SemaphoreType.DMA,
    ],
)


def _psum_body(p_ref, o_ref):
    o_ref[...] = p_ref[0] + p_ref[1]


_psum = pl.pallas_call(
    _psum_body,
    grid=(N // _BN,),
    in_specs=[pl.BlockSpec((NC, _BN, D), lambda i: (0, i, 0))],
    out_specs=pl.BlockSpec((_BN, D), lambda i: (i, 0)),
    out_shape=jax.ShapeDtypeStruct((N, D), jnp.float32),
)


@jax.jit
def kernel(node, edge, seg_i, idx_j, W_node, W_e1, b_e1, W_e2, b_e2):
    msg = _mm_node(node, W_node)
    edge_pad = jnp.pad(edge, ((0, E_PAD - E), (0, 0)))
    h = _edge_proj(edge_pad, W_e1, b_e1.reshape(1, D), W_e2, b_e2.reshape(1, D))
    packed = jnp.pad((idx_j << IDX_SHIFT) | seg_i, (0, E_PAD - E))
    pk2 = packed.reshape(R, 2 * CH)
    partials = _sc_gather_scatter(msg, h, pk2)
    return _psum(partials)


# speedup vs baseline: 1.6099x; 1.5426x over previous
import jax
import jax.numpy as jnp
from jax import lax
from jax.experimental import pallas as pl
from jax.experimental.pallas import tpu as pltpu
from jax.experimental.pallas import tpu_sc as plsc

N = 10000
E = 320000
D = 128
DE = 16
NC, NS, L = 2, 16, 16

CH = 80
NCHUNKS = E // CH
CH_A = 176
CH_B = NCHUNKS // NS - CH_A
ZROWS = 632
ZLAST = N - ZROWS
IDX_SHIFT = 14
NBUF = 2


def _leaky(x):
    return jnp.where(x >= 0, x, x * jnp.float32(0.01))


_BN = 2000


def _mm_node_body(x_ref, w_ref, o_ref):
    o_ref[...] = jnp.dot(x_ref[...], w_ref[...], preferred_element_type=jnp.float32)


_mm_node = pl.pallas_call(
    _mm_node_body,
    grid=(N // _BN,),
    in_specs=[
        pl.BlockSpec((_BN, D), lambda i: (i, 0)),
        pl.BlockSpec((D, D), lambda i: (0, 0)),
    ],
    out_specs=pl.BlockSpec((_BN, D), lambda i: (i, 0)),
    out_shape=jax.ShapeDtypeStruct((N, D), jnp.float32),
)


_BE = 2000


def _edge_body(e_ref, w1_ref, b1_ref, w2_ref, b2_ref, o_ref):
    h = jnp.dot(e_ref[...], w1_ref[...], preferred_element_type=jnp.float32)
    h = _leaky(h + b1_ref[...])
    h = jnp.dot(h, w2_ref[...], preferred_element_type=jnp.float32)
    o_ref[...] = _leaky(h + b2_ref[...])


_edge_proj = pl.pallas_call(
    _edge_body,
    grid=(E // _BE,),
    in_specs=[
        pl.BlockSpec((_BE, DE), lambda i: (i, 0)),
        pl.BlockSpec((DE, D), lambda i: (0, 0)),
        pl.BlockSpec((1, D), lambda i: (0, 0)),
        pl.BlockSpec((D, D), lambda i: (0, 0)),
        pl.BlockSpec((1, D), lambda i: (0, 0)),
    ],
    out_specs=pl.BlockSpec((_BE, D), lambda i: (i, 0)),
    out_shape=jax.ShapeDtypeStruct((E, D), jnp.float32),
)


def _sc_body(msg_hbm, h_hbm, pk_hbm, out_hbm, *scr):
    pkr = scr[0]
    idxrv = scr[1]
    segrv = scr[2]
    rows = list(scr[3:3 + NBUF])
    hvb = list(scr[3 + NBUF:3 + 2 * NBUF])
    acc_sh = scr[3 + 2 * NBUF]
    gsem = list(scr[4 + 2 * NBUF:4 + 3 * NBUF])
    hsem = list(scr[4 + 3 * NBUF:4 + 4 * NBUF])
    ssem = list(scr[4 + 4 * NBUF:4 + 5 * NBUF])
    pksem = list(scr[4 + 5 * NBUF:4 + 6 * NBUF])

    c = lax.axis_index("c")
    s = lax.axis_index("s")
    w = c * NS + s

    def _z(i, _):
        zero = jnp.zeros((L,), jnp.float32)
        for k in range(D // L):
            rows[0][i, pl.ds(k * L, L)] = zero
        return 0
    lax.fori_loop(0, CH, _z, 0)
    zbase = jnp.minimum(s * ZROWS, ZLAST)
    for t in range(ZROWS // CH):
        pltpu.sync_copy(rows[0], acc_sh.at[pl.ds(zbase + t * CH, CH)])
    zrem = ZROWS % CH
    if zrem:
        pltpu.sync_copy(rows[0].at[pl.ds(0, zrem)],
                        acc_sh.at[pl.ds(zbase + (ZROWS // CH) * CH, zrem)])
    plsc.subcore_barrier()

    nchunks = jnp.where(c == 0, CH_A, CH_B)
    cbase = jnp.where(c == 0, s * CH_A, NS * CH_A + s * CH_B)

    def _pkload(t, b):
        pltpu.async_copy(pk_hbm.at[pl.ds((cbase + t) * CH, CH)],
                         pkr.at[b], pksem[b])

    def _prep(t, b):
        pltpu.make_async_copy(pk_hbm.at[pl.ds(0, CH)],
                              pkr.at[b], pksem[b]).wait()

        def _unpack(k, _):
            pk = pkr[b, pl.ds(k * L, L)]
            sl = pl.ds(k * L, L)
            idxrv[b, sl] = lax.shift_right_logical(pk, IDX_SHIFT)
            segrv[b, sl] = lax.bitwise_and(pk, (1 << IDX_SHIFT) - 1)
            return 0
        lax.fori_loop(0, CH // L, _unpack, 0)
        pltpu.async_copy(msg_hbm.at[idxrv.at[b]], rows[b], gsem[b])
        pltpu.async_copy(h_hbm.at[pl.ds((cbase + t) * CH, CH)],
                         hvb[b], hsem[b])

        @pl.when(t + NBUF < nchunks)
        def _():
            _pkload(t + NBUF, b)

    for b in range(NBUF):
        _pkload(jnp.int32(b), b)
    for b in range(NBUF):
        _prep(jnp.int32(b), b)

    def _work(t, b):
        pltpu.make_async_copy(msg_hbm.at[idxrv.at[b]],
                              rows[b], gsem[b]).wait()
        pltpu.make_async_copy(h_hbm.at[pl.ds(0, CH)],
                              hvb[b], hsem[b]).wait()

        def _mul(i, _):
            for k in range(D // L):
                sl = pl.ds(k * L, L)
                rows[b][i, sl] = rows[b][i, sl] * hvb[b][i, sl]
            return 0
        lax.fori_loop(0, CH, _mul, 0)
        pltpu.async_copy(rows[b], acc_sh.at[segrv.at[b]],
                         ssem[b], add=True)
        t2 = t + NBUF

        @pl.when(t2 < nchunks)
        def _():
            pltpu.make_async_copy(rows[b], acc_sh.at[segrv.at[b]],
                                  ssem[b]).wait()
            _prep(t2, b)

    def _step(g, _):
        for b in range(NBUF):
            _work(g * NBUF + b, b)
        return 0
    lax.fori_loop(0, jnp.where(c == 0, CH_A // NBUF, CH_B // NBUF), _step, 0)

    for b in range(NBUF):
        pltpu.make_async_copy(rows[b], acc_sh.at[segrv.at[b]],
                              ssem[b]).wait()
    plsc.subcore_barrier()
    pltpu.sync_copy(acc_sh.at[pl.ds(zbase, ZROWS)],
                    out_hbm.at[c, pl.ds(zbase, ZROWS)])


_sc_gather_scatter = pl.kernel(
    _sc_body,
    out_type=jax.ShapeDtypeStruct((NC, N, D), jnp.float32),
    mesh=plsc.VectorSubcoreMesh(
        core_axis_name="c", subcore_axis_name="s",
        num_cores=NC, num_subcores=NS),
    scratch_types=(
        [pltpu.VMEM((NBUF, CH), jnp.int32),
         pltpu.VMEM((NBUF, CH), jnp.int32),
         pltpu.VMEM((NBUF, CH), jnp.int32)]
        + [pltpu.VMEM((CH, D), jnp.float32) for _ in range(2 * NBUF)]
        + [pltpu.VMEM_SHARED((N, D), jnp.float32)]
        + [pltpu.SemaphoreType.DMA for _ in range(4 * NBUF)]
    ),
)


def _psum_body(p_ref, o_ref):
    o_ref[...] = p_ref[0] + p_ref[1]


_psum = pl.pallas_call(
    _psum_body,
    grid=(N // _BN,),
    in_specs=[pl.BlockSpec((NC, _BN, D), lambda i: (0, i, 0))],
    out_specs=pl.BlockSpec((_BN, D), lambda i: (i, 0)),
    out_shape=jax.ShapeDtypeStruct((N, D), jnp.float32),
)


@jax.jit
def kernel(node, edge, seg_i, idx_j, W_node, W_e1, b_e1, W_e2, b_e2):
    msg = _mm_node(node, W_node)
    h = _edge_proj(edge, W_e1, b_e1.reshape(1, D), W_e2, b_e2.reshape(1, D))
    pk1 = (idx_j << IDX_SHIFT) | seg_i
    partials = _sc_gather_scatter(msg, h, pk1)
    return _psum(partials)


# speedup vs baseline: 1.8857x; 1.1713x over previous
import jax
import jax.numpy as jnp
from jax import lax
from jax.experimental import pallas as pl
from jax.experimental.pallas import tpu as pltpu
from jax.experimental.pallas import tpu_sc as plsc

N = 10000
E = 320000
D = 128
DE = 16
NC, NS, L = 2, 16, 16

CH = 80
WORDS = CH // 2
NCHUNKS = E // CH
CH_A = 176
CH_B = NCHUNKS // NS - CH_A
ZROWS = 632
ZLAST = N - ZROWS
IDX_SHIFT = 14
NBUF = 2


def _leaky(x):
    return jnp.where(x >= 0, x, x * jnp.float32(0.01))


_BN = 2000


def _mm_node_body(x_ref, w_ref, o_ref):
    o_ref[...] = jnp.dot(x_ref[...], w_ref[...], preferred_element_type=jnp.float32)


_mm_node = pl.pallas_call(
    _mm_node_body,
    grid=(N // _BN,),
    in_specs=[
        pl.BlockSpec((_BN, D), lambda i: (i, 0)),
        pl.BlockSpec((D, D), lambda i: (0, 0)),
    ],
    out_specs=pl.BlockSpec((_BN, D), lambda i: (i, 0)),
    out_shape=jax.ShapeDtypeStruct((N, D), jnp.float32),
)


_BE = 2000


def _mlp(e, w1, b1, w2, b2):
    h = jnp.dot(e, w1, preferred_element_type=jnp.float32)
    h = _leaky(h + b1)
    h = jnp.dot(h, w2, preferred_element_type=jnp.float32)
    return _leaky(h + b2)


def _edge_body(ea_ref, eb_ref, w1_ref, b1_ref, w2_ref, b2_ref, o_ref):
    ha = _mlp(ea_ref[...], w1_ref[...], b1_ref[...], w2_ref[...], b2_ref[...])
    hb = _mlp(eb_ref[...], w1_ref[...], b1_ref[...], w2_ref[...], b2_ref[...])
    ua = lax.bitcast_convert_type(ha.astype(jnp.bfloat16), jnp.uint16).astype(jnp.uint32)
    ub = lax.bitcast_convert_type(hb.astype(jnp.bfloat16), jnp.uint16).astype(jnp.uint32)
    o_ref[...] = lax.bitcast_convert_type(ua | (ub << 16), jnp.int32)


_NB2 = E // 2 // _BE

_edge_proj = pl.pallas_call(
    _edge_body,
    grid=(_NB2,),
    in_specs=[
        pl.BlockSpec((_BE, DE), lambda i: (i, 0)),
        pl.BlockSpec((_BE, DE), lambda i: (i + _NB2, 0)),
        pl.BlockSpec((DE, D), lambda i: (0, 0)),
        pl.BlockSpec((1, D), lambda i: (0, 0)),
        pl.BlockSpec((D, D), lambda i: (0, 0)),
        pl.BlockSpec((1, D), lambda i: (0, 0)),
    ],
    out_specs=pl.BlockSpec((_BE, D), lambda i: (i, 0)),
    out_shape=jax.ShapeDtypeStruct((E // 2, D), jnp.int32),
)


def _sc_body(msg_hbm, h_hbm, pk_hbm, out_hbm, *scr):
    pkr = scr[0]
    idxrv = scr[1]
    segrv = scr[2]
    rows = list(scr[3:3 + NBUF])
    hvb = list(scr[3 + NBUF:3 + 2 * NBUF])
    acc_sh = scr[3 + 2 * NBUF]
    gsem = list(scr[4 + 2 * NBUF:4 + 3 * NBUF])
    hsem = list(scr[4 + 3 * NBUF:4 + 4 * NBUF])
    ssem = list(scr[4 + 4 * NBUF:4 + 5 * NBUF])
    pksem = list(scr[4 + 5 * NBUF:4 + 6 * NBUF])

    c = lax.axis_index("c")
    s = lax.axis_index("s")
    w = c * NS + s

    def _z(i, _):
        zero = jnp.zeros((L,), jnp.float32)
        for k in range(D // L):
            rows[0][i, pl.ds(k * L, L)] = zero
        return 0
    lax.fori_loop(0, CH, _z, 0)
    zbase = jnp.minimum(s * ZROWS, ZLAST)
    for t in range(ZROWS // CH):
        pltpu.sync_copy(rows[0], acc_sh.at[pl.ds(zbase + t * CH, CH)])
    zrem = ZROWS % CH
    if zrem:
        pltpu.sync_copy(rows[0].at[pl.ds(0, zrem)],
                        acc_sh.at[pl.ds(zbase + (ZROWS // CH) * CH, zrem)])
    plsc.subcore_barrier()

    nchunks = jnp.where(c == 0, CH_A, CH_B)
    cbase = jnp.where(c == 0, s * CH_A, NS * CH_A + s * CH_B)

    def _pkload(t, b):
        pltpu.async_copy(pk_hbm.at[pl.ds((cbase + t) * CH, CH)],
                         pkr.at[b], pksem[b])

    def _prep(t, b):
        pltpu.make_async_copy(pk_hbm.at[pl.ds(0, CH)],
                              pkr.at[b], pksem[b]).wait()

        def _unpack(k, _):
            pk = pkr[b, pl.ds(k * L, L)]
            sl = pl.ds(k * L, L)
            idxrv[b, sl] = lax.shift_right_logical(pk, IDX_SHIFT)
            segrv[b, sl] = lax.bitwise_and(pk, (1 << IDX_SHIFT) - 1)
            return 0
        lax.fori_loop(0, CH // L, _unpack, 0)
        pltpu.async_copy(msg_hbm.at[idxrv.at[b]], rows[b], gsem[b])
        pltpu.async_copy(h_hbm.at[pl.ds((cbase + t) * WORDS, WORDS)],
                         hvb[b], hsem[b])

        @pl.when(t + NBUF < nchunks)
        def _():
            _pkload(t + NBUF, b)

    for b in range(NBUF):
        _pkload(jnp.int32(b), b)
    for b in range(NBUF):
        _prep(jnp.int32(b), b)

    def _work(t, b):
        pltpu.make_async_copy(msg_hbm.at[idxrv.at[b]],
                              rows[b], gsem[b]).wait()
        pltpu.make_async_copy(h_hbm.at[pl.ds(0, WORDS)],
                              hvb[b], hsem[b]).wait()

        def _mul(i, _):
            for k in range(D // L):
                sl = pl.ds(k * L, L)
                hb2 = plsc.bitcast(hvb[b][i, sl], jnp.bfloat16)
                h0, h1 = plsc.unpack(hb2, format=plsc.PackFormat.INTERLEAVED)
                rows[b][i, sl] = rows[b][i, sl] * h0
                rows[b][i + WORDS, sl] = rows[b][i + WORDS, sl] * h1
            return 0
        lax.fori_loop(0, WORDS, _mul, 0)
        pltpu.async_copy(rows[b], acc_sh.at[segrv.at[b]],
                         ssem[b], add=True)
        t2 = t + NBUF

        @pl.when(t2 < nchunks)
        def _():
            pltpu.make_async_copy(rows[b], acc_sh.at[segrv.at[b]],
                                  ssem[b]).wait()
            _prep(t2, b)

    def _step(g, _):
        for b in range(NBUF):
            _work(g * NBUF + b, b)
        return 0
    lax.fori_loop(0, jnp.where(c == 0, CH_A // NBUF, CH_B // NBUF), _step, 0)

    for b in range(NBUF):
        pltpu.make_async_copy(rows[b], acc_sh.at[segrv.at[b]],
                              ssem[b]).wait()
    plsc.subcore_barrier()
    pltpu.sync_copy(acc_sh.at[pl.ds(zbase, ZROWS)],
                    out_hbm.at[c, pl.ds(zbase, ZROWS)])


_sc_gather_scatter = pl.kernel(
    _sc_body,
    out_type=jax.ShapeDtypeStruct((NC, N, D), jnp.float32),
    mesh=plsc.VectorSubcoreMesh(
        core_axis_name="c", subcore_axis_name="s",
        num_cores=NC, num_subcores=NS),
    compiler_params=pltpu.CompilerParams(needs_layout_passes=False),
    scratch_types=(
        [pltpu.VMEM((NBUF, CH), jnp.int32),
         pltpu.VMEM((NBUF, CH), jnp.int32),
         pltpu.VMEM((NBUF, CH), jnp.int32)]
        + [pltpu.VMEM((CH, D), jnp.float32) for _ in range(NBUF)]
        + [pltpu.VMEM((WORDS, D), jnp.int32) for _ in range(NBUF)]
        + [pltpu.VMEM_SHARED((N, D), jnp.float32)]
        + [pltpu.SemaphoreType.DMA for _ in range(4 * NBUF)]
    ),
)


def _psum_body(p_ref, o_ref):
    o_ref[...] = p_ref[0] + p_ref[1]


_psum = pl.pallas_call(
    _psum_body,
    grid=(N // _BN,),
    in_specs=[pl.BlockSpec((NC, _BN, D), lambda i: (0, i, 0))],
    out_specs=pl.BlockSpec((_BN, D), lambda i: (i, 0)),
    out_shape=jax.ShapeDtypeStruct((N, D), jnp.float32),
)


@jax.jit
def kernel(node, edge, seg_i, idx_j, W_node, W_e1, b_e1, W_e2, b_e2):
    msg = _mm_node(node, W_node)
    h = _edge_proj(edge, edge, W_e1, b_e1.reshape(1, D), W_e2, b_e2.reshape(1, D))
    packed = (idx_j << IDX_SHIFT) | seg_i
    pk1 = jnp.concatenate(
        [packed[:E // 2].reshape(NCHUNKS, WORDS),
         packed[E // 2:].reshape(NCHUNKS, WORDS)], axis=1).reshape(E)
    partials = _sc_gather_scatter(msg, h, pk1)
    return _psum(partials)


# speedup vs baseline: 2.1938x; 1.1634x over previous
import jax
import jax.numpy as jnp
from jax import lax
from jax.experimental import pallas as pl
from jax.experimental.pallas import tpu as pltpu
from jax.experimental.pallas import tpu_sc as plsc

N = 10000
E = 320000
D = 128
DE = 16
NC, NS, L = 2, 16, 16

CH = 80
WORDS = CH // 2
NCHUNKS = E // CH
CH_A = 130
CH_B = NCHUNKS // NS - CH_A
ZROWS = 632
ZLAST = N - ZROWS
IDX_SHIFT = 14
NBUF = 2


def _leaky(x):
    return jnp.where(x >= 0, x, x * jnp.float32(0.01))


_BN = 2000


def _mm_node_body(x_ref, w_ref, o_ref):
    o_ref[...] = jnp.dot(x_ref[...], w_ref[...], preferred_element_type=jnp.float32)


_mm_node = pl.pallas_call(
    _mm_node_body,
    grid=(N // _BN,),
    in_specs=[
        pl.BlockSpec((_BN, D), lambda i: (i, 0)),
        pl.BlockSpec((D, D), lambda i: (0, 0)),
    ],
    out_specs=pl.BlockSpec((_BN, D), lambda i: (i, 0)),
    out_shape=jax.ShapeDtypeStruct((N, D), jnp.float32),
)


_BE = 2000


def _mlp(e, w1, b1, w2, b2):
    h = jnp.dot(e, w1, preferred_element_type=jnp.float32)
    h = _leaky(h + b1)
    h = jnp.dot(h, w2, preferred_element_type=jnp.float32)
    return _leaky(h + b2)


def _edge_body(ea_ref, eb_ref, w1_ref, b1_ref, w2_ref, b2_ref, o_ref):
    ha = _mlp(ea_ref[...], w1_ref[...], b1_ref[...], w2_ref[...], b2_ref[...])
    hb = _mlp(eb_ref[...], w1_ref[...], b1_ref[...], w2_ref[...], b2_ref[...])
    ua = lax.bitcast_convert_type(ha.astype(jnp.bfloat16), jnp.uint16).astype(jnp.uint32)
    ub = lax.bitcast_convert_type(hb.astype(jnp.bfloat16), jnp.uint16).astype(jnp.uint32)
    o_ref[...] = lax.bitcast_convert_type(ua | (ub << 16), jnp.int32)


_NB2 = E // 2 // _BE

_edge_proj = pl.pallas_call(
    _edge_body,
    grid=(_NB2,),
    in_specs=[
        pl.BlockSpec((_BE, DE), lambda i: (i, 0)),
        pl.BlockSpec((_BE, DE), lambda i: (i + _NB2, 0)),
        pl.BlockSpec((DE, D), lambda i: (0, 0)),
        pl.BlockSpec((1, D), lambda i: (0, 0)),
        pl.BlockSpec((D, D), lambda i: (0, 0)),
        pl.BlockSpec((1, D), lambda i: (0, 0)),
    ],
    out_specs=pl.BlockSpec((_BE, D), lambda i: (i, 0)),
    out_shape=jax.ShapeDtypeStruct((E // 2, D), jnp.int32),
)


def _sc_body(msg_hbm, h_hbm, pk_hbm, out_hbm, *scr):
    pkr = scr[0]
    idxrv = scr[1]
    segrv = scr[2]
    rows = list(scr[3:3 + NBUF])
    hvb = list(scr[3 + NBUF:3 + 2 * NBUF])
    acc_sh = scr[3 + 2 * NBUF]
    gsem = list(scr[4 + 2 * NBUF:4 + 3 * NBUF])
    hsem = list(scr[4 + 3 * NBUF:4 + 4 * NBUF])
    ssem = list(scr[4 + 4 * NBUF:4 + 5 * NBUF])
    pksem = list(scr[4 + 5 * NBUF:4 + 6 * NBUF])

    c = lax.axis_index("c")
    s = lax.axis_index("s")
    w = c * NS + s

    def _z(i, _):
        zero = jnp.zeros((L,), jnp.float32)
        for k in range(D // L):
            rows[0][i, pl.ds(k * L, L)] = zero
        return 0
    lax.fori_loop(0, CH, _z, 0)
    zbase = jnp.minimum(s * ZROWS, ZLAST)
    for t in range(ZROWS // CH):
        pltpu.sync_copy(rows[0], acc_sh.at[pl.ds(zbase + t * CH, CH)])
    zrem = ZROWS % CH
    if zrem:
        pltpu.sync_copy(rows[0].at[pl.ds(0, zrem)],
                        acc_sh.at[pl.ds(zbase + (ZROWS // CH) * CH, zrem)])
    plsc.subcore_barrier()

    nchunks = jnp.where(c == 0, CH_A, CH_B)
    cbase = jnp.where(c == 0, s * CH_A, NS * CH_A + s * CH_B)

    def _pkload(t, b):
        pltpu.async_copy(pk_hbm.at[pl.ds((cbase + t) * WORDS, WORDS)],
                         pkr.at[b, pl.ds(0, WORDS)], pksem[b])
        pltpu.async_copy(pk_hbm.at[pl.ds(E // 2 + (cbase + t) * WORDS, WORDS)],
                         pkr.at[b, pl.ds(WORDS, WORDS)], pksem[b])

    def _prep(t, b):
        pltpu.make_async_copy(pk_hbm.at[pl.ds(0, WORDS)],
                              pkr.at[b, pl.ds(0, WORDS)], pksem[b]).wait()
        pltpu.make_async_copy(pk_hbm.at[pl.ds(0, WORDS)],
                              pkr.at[b, pl.ds(WORDS, WORDS)], pksem[b]).wait()

        def _unpack(k, _):
            pk = pkr[b, pl.ds(k * L, L)]
            sl = pl.ds(k * L, L)
            idxrv[b, sl] = lax.shift_right_logical(pk, IDX_SHIFT)
            segrv[b, sl] = lax.bitwise_and(pk, (1 << IDX_SHIFT) - 1)
            return 0
        lax.fori_loop(0, CH // L, _unpack, 0)
        pltpu.async_copy(msg_hbm.at[idxrv.at[b]], rows[b], gsem[b])
        pltpu.async_copy(h_hbm.at[pl.ds((cbase + t) * WORDS, WORDS)],
                         hvb[b], hsem[b])

        @pl.when(t + NBUF < nchunks)
        def _():
            _pkload(t + NBUF, b)

    for b in range(NBUF):
        _pkload(jnp.int32(b), b)
    for b in range(NBUF):
        _prep(jnp.int32(b), b)

    def _work(t, b):
        pltpu.make_async_copy(msg_hbm.at[idxrv.at[b]],
                              rows[b], gsem[b]).wait()
        pltpu.make_async_copy(h_hbm.at[pl.ds(0, WORDS)],
                              hvb[b], hsem[b]).wait()

        def _mul(i, _):
            for k in range(D // L):
                sl = pl.ds(k * L, L)
                hb2 = plsc.bitcast(hvb[b][i, sl], jnp.bfloat16)
                h0, h1 = plsc.unpack(hb2, format=plsc.PackFormat.INTERLEAVED)
                rows[b][i, sl] = rows[b][i, sl] * h0
                rows[b][i + WORDS, sl] = rows[b][i + WORDS, sl] * h1
            return 0
        lax.fori_loop(0, WORDS, _mul, 0)
        pltpu.async_copy(rows[b], acc_sh.at[segrv.at[b]],
                         ssem[b], add=True)
        t2 = t + NBUF

        @pl.when(t2 < nchunks)
        def _():
            pltpu.make_async_copy(rows[b], acc_sh.at[segrv.at[b]],
                                  ssem[b]).wait()
            _prep(t2, b)

    def _step(g, _):
        for b in range(NBUF):
            _work(g * NBUF + b, b)
        return 0
    lax.fori_loop(0, jnp.where(c == 0, CH_A // NBUF, CH_B // NBUF), _step, 0)

    for b in range(NBUF):
        pltpu.make_async_copy(rows[b], acc_sh.at[segrv.at[b]],
                              ssem[b]).wait()
    plsc.subcore_barrier()
    pltpu.sync_copy(acc_sh.at[pl.ds(zbase, ZROWS)],
                    out_hbm.at[c, pl.ds(zbase, ZROWS)])


_sc_gather_scatter = pl.kernel(
    _sc_body,
    out_type=jax.ShapeDtypeStruct((NC, N, D), jnp.float32),
    mesh=plsc.VectorSubcoreMesh(
        core_axis_name="c", subcore_axis_name="s",
        num_cores=NC, num_subcores=NS),
    compiler_params=pltpu.CompilerParams(needs_layout_passes=False),
    scratch_types=(
        [pltpu.VMEM((NBUF, CH), jnp.int32),
         pltpu.VMEM((NBUF, CH), jnp.int32),
         pltpu.VMEM((NBUF, CH), jnp.int32)]
        + [pltpu.VMEM((CH, D), jnp.float32) for _ in range(NBUF)]
        + [pltpu.VMEM((WORDS, D), jnp.int32) for _ in range(NBUF)]
        + [pltpu.VMEM_SHARED((N, D), jnp.float32)]
        + [pltpu.SemaphoreType.DMA for _ in range(4 * NBUF)]
    ),
)


def _psum_body(p_ref, o_ref):
    o_ref[...] = p_ref[0] + p_ref[1]


_psum = pl.pallas_call(
    _psum_body,
    grid=(N // _BN,),
    in_specs=[pl.BlockSpec((NC, _BN, D), lambda i: (0, i, 0))],
    out_specs=pl.BlockSpec((_BN, D), lambda i: (i, 0)),
    out_shape=jax.ShapeDtypeStruct((N, D), jnp.float32),
)


@jax.jit
def kernel(node, edge, seg_i, idx_j, W_node, W_e1, b_e1, W_e2, b_e2):
    msg = _mm_node(node, W_node)
    h = _edge_proj(edge, edge, W_e1, b_e1.reshape(1, D), W_e2, b_e2.reshape(1, D))
    pk1 = (idx_j << IDX_SHIFT) | seg_i
    partials = _sc_gather_scatter(msg, h, pk1)
    return _psum(partials)
